# baseline (speedup 1.0000x reference)
import jax
import jax.numpy as jnp
import numpy as np
from jax.experimental import pallas as pl
from jax.experimental.pallas import tpu as pltpu

_T, _D = 16, 128
_DG = 128
_G, _EPG = 4, 16
_HID = _G * _EPG
_NH = 8
_HD = _EPG // _NH
_K = 8
_B = 1024

_f32 = jnp.float32


def _dot(a, b):
    return jnp.dot(a, b, preferred_element_type=_f32)


def _dot_t(a, b):
    return jax.lax.dot_general(a, b, (((1,), (1,)), ((), ())),
                               preferred_element_type=_f32)


def _prep_body(wq_ref, wk_ref, wv_ref, wo_ref, wexp_ref, bexp_ref,
               bq_ref, bk_ref, bv_ref, gm_ref, se_ref, so_ref,
               set_ref, sot_ref, biga_ref, bigc_ref, bigwo_ref):
    wexp = wexp_ref[...]
    gm = gm_ref[...]
    for j, (wref, bref) in enumerate(((wq_ref, bq_ref), (wk_ref, bk_ref),
                                      (wv_ref, bv_ref))):
        wf = wref[...]
        bd = jnp.concatenate([wf, wf, wf, wf], axis=1) * gm
        prod = _dot(bd, wexp)
        biga_ref[j * 64:j * 64 + 32, :] = _dot(set_ref[...], prod)
        biga_ref[j * 64 + 32:(j + 1) * 64, :] = _dot(sot_ref[...], prod)
        crow = _dot_t(bexp_ref[...], bd) + bref[...]
        bigc_ref[2 * j:2 * j + 1, :] = _dot(crow, se_ref[...])
        bigc_ref[2 * j + 1:2 * j + 2, :] = _dot(crow, so_ref[...])
    wo = wo_ref[...]
    bdo = jnp.concatenate([wo, wo, wo, wo], axis=1) * gm
    bigwo_ref[0:64, :] = _dot(bdo, se_ref[...])
    bigwo_ref[64:128, :] = _dot(bdo, so_ref[...])


def _body(price_ref, news_ref, mask_ref, wr_ref, br_ref, wg_ref, bg_ref,
          biga_ref, bigc_ref, bigwo_ref, segs_ref, segt_ref, bo_ref,
          pred_ref, rw_ref, h_ref, idx_ref):
    x = price_ref[...] + news_ref[...] * mask_ref[...][:, :, None]
    pooled = jnp.sum(x, axis=1) * (1.0 / _T)
    hr = jnp.tanh(_dot(pooled, wr_ref[...]) + br_ref[...])
    h = _dot(hr, wg_ref[...]) + bg_ref[...]
    h_ref[...] = h

    qe = _dot_t(h, biga_ref[0:32, :]) + bigc_ref[0:1, :]
    qo = _dot_t(h, biga_ref[32:64, :]) + bigc_ref[1:2, :]
    ke = _dot_t(h, biga_ref[64:96, :]) + bigc_ref[2:3, :]
    ko = _dot_t(h, biga_ref[96:128, :]) + bigc_ref[3:4, :]
    ve = _dot_t(h, biga_ref[128:160, :]) + bigc_ref[4:5, :]
    vo = _dot_t(h, biga_ref[160:192, :]) + bigc_ref[5:6, :]
    p00 = _dot(qe * ke, segs_ref[...])
    p01 = _dot(qe * ko, segs_ref[...])
    p10 = _dot(qo * ke, segs_ref[...])
    p11 = _dot(qo * ko, segs_ref[...])
    m0 = jnp.maximum(p00, p01)
    e00 = jnp.exp(p00 - m0)
    e01 = jnp.exp(p01 - m0)
    m1 = jnp.maximum(p10, p11)
    e10 = jnp.exp(p10 - m1)
    e11 = jnp.exp(p11 - m1)
    z0 = e00 + e01
    z1 = e10 + e11
    a00 = e00 / z0
    a01 = e01 / z0
    a10 = e10 / z1
    a11 = e11 / z1
    av_e = _dot(a00, segt_ref[...]) * ve + _dot(a01, segt_ref[...]) * vo
    av_o = _dot(a10, segt_ref[...]) * ve + _dot(a11, segt_ref[...]) * vo
    agg = (_dot_t(av_e, bigwo_ref[0:64, :]) +
           _dot_t(av_o, bigwo_ref[64:128, :]) + bo_ref[...])

    iota = jax.lax.broadcasted_iota(jnp.int32, h.shape, 1)
    cur = h
    sel = jnp.zeros(h.shape, jnp.bool_)
    mtop = None
    idx_cols = []
    for k in range(_K):
        m = jnp.max(cur, axis=1, keepdims=True)
        if k == 0:
            mtop = m
        idx = jnp.min(jnp.where(cur == m, iota, _HID), axis=1, keepdims=True)
        idx_cols.append(idx)
        chosen = iota == idx
        sel = jnp.logical_or(sel, chosen)
        cur = jnp.where(chosen, -jnp.inf, cur)
    idx_ref[...] = jnp.concatenate(idx_cols, axis=1)
    ex = jnp.where(sel, jnp.exp(h - mtop), 0.0)
    rw = ex / jnp.sum(ex, axis=1, keepdims=True)
    rw_ref[...] = rw
    pred_ref[...] = jnp.sum(agg * rw, axis=1, keepdims=True)


def kernel(price_feature, news_feature, news_mask, W_r, b_r, W_g, b_g,
           W_exp, b_exp, Wq, bq, Wk, bk, Wv, bv, Wo, bo):
    n = price_feature.shape[0]

    se = np.zeros((_HID, _G * _NH), np.float32)
    so = np.zeros((_HID, _G * _NH), np.float32)
    seg = np.zeros((_G * _NH, _G), np.float32)
    for g in range(_G):
        for hh in range(_NH):
            se[g * _EPG + 2 * hh, g * _NH + hh] = 1.0
            so[g * _EPG + 2 * hh + 1, g * _NH + hh] = 1.0
            seg[g * _NH + hh, g] = 1.0
    gm = np.zeros((_HID, _HID), np.float32)
    for g in range(_G):
        gm[g * _EPG:(g + 1) * _EPG, g * _EPG:(g + 1) * _EPG] = 1.0
    segs = jnp.asarray(seg / np.sqrt(np.float32(_HD)))
    segt = jnp.asarray(seg.T)

    def full(shape):
        return pl.BlockSpec(shape, lambda i: tuple(0 for _ in shape))

    biga, bigc, bigwo = pl.pallas_call(
        _prep_body,
        grid=(1,),
        in_specs=[full((_HID, _EPG))] * 4 + [
            full((_HID, _HID)), full((1, _HID)),
            full((1, _HID)), full((1, _HID)), full((1, _HID)),
            full((_HID, _HID)),
            full((_HID, 32)), full((_HID, 32)),
            full((32, _HID)), full((32, _HID)),
        ],
        out_specs=(full((192, _HID)), full((8, 32)), full((128, 32))),
        out_shape=(
            jax.ShapeDtypeStruct((192, _HID), _f32),
            jax.ShapeDtypeStruct((8, 32), _f32),
            jax.ShapeDtypeStruct((128, 32), _f32),
        ),
    )(Wq.reshape(_HID, _EPG), Wk.reshape(_HID, _EPG),
      Wv.reshape(_HID, _EPG), Wo.reshape(_HID, _EPG),
      W_exp.reshape(_HID, _HID), b_exp.reshape(1, -1),
      bq.reshape(1, -1), bk.reshape(1, -1), bv.reshape(1, -1),
      jnp.asarray(gm), jnp.asarray(se), jnp.asarray(so),
      jnp.asarray(se.T), jnp.asarray(so.T))

    b = _B if n % _B == 0 else n
    grid = (n // b,)

    outs = pl.pallas_call(
        _body,
        grid=grid,
        in_specs=[
            pl.BlockSpec((b, _T, _D), lambda i: (i, 0, 0)),
            pl.BlockSpec((b, _T, _D), lambda i: (i, 0, 0)),
            pl.BlockSpec((b, _T), lambda i: (i, 0)),
            full((_D, _DG)), full((1, _DG)), full((_DG, _HID)), full((1, _HID)),
            full((192, _HID)), full((8, 32)), full((128, 32)),
            full((_G * _NH, _G)), full((_G, _G * _NH)), full((1, _HID)),
        ],
        out_specs=(
            pl.BlockSpec((b, 1), lambda i: (i, 0)),
            pl.BlockSpec((b, _HID), lambda i: (i, 0)),
            pl.BlockSpec((b, _HID), lambda i: (i, 0)),
            pl.BlockSpec((b, _K), lambda i: (i, 0)),
        ),
        out_shape=(
            jax.ShapeDtypeStruct((n, 1), _f32),
            jax.ShapeDtypeStruct((n, _HID), _f32),
            jax.ShapeDtypeStruct((n, _HID), _f32),
            jax.ShapeDtypeStruct((n, _K), jnp.int32),
        ),
        compiler_params=pltpu.CompilerParams(
            dimension_semantics=("arbitrary",),
            vmem_limit_bytes=100 * 1024 * 1024),
    )(price_feature, news_feature, news_mask,
      W_r, b_r.reshape(1, -1), W_g, b_g.reshape(1, -1),
      biga, bigc, bigwo, segs, segt, bo.reshape(1, -1))

    preds, rw, h, idx = outs
    return preds.reshape(n), rw, h, idx, rw


# speedup vs baseline: 1.7449x; 1.7449x over previous
import jax
import jax.numpy as jnp
import numpy as np
from jax.experimental import pallas as pl
from jax.experimental.pallas import tpu as pltpu

_T, _D = 16, 128
_DG = 128
_G, _EPG = 4, 16
_HID = _G * _EPG
_NH = 8
_HD = _EPG // _NH
_K = 8
_B = 1024

_f32 = jnp.float32


def _dot(a, b):
    return jnp.dot(a, b, preferred_element_type=_f32)


def _dot_t(a, b):
    return jax.lax.dot_general(a, b, (((1,), (1,)), ((), ())),
                               preferred_element_type=_f32)


def _dot_tl(a, b):
    return jax.lax.dot_general(a, b, (((0,), (0,)), ((), ())),
                               preferred_element_type=_f32)


def _prep_body(wq_ref, wk_ref, wv_ref, wo_ref, wexp_ref, bexp_ref,
               bq_ref, bk_ref, bv_ref, bo_ref, gm_ref, se_ref, so_ref,
               biga_ref, bigwo_ref):
    wexp = wexp_ref[...]
    gm = gm_ref[...]
    se = se_ref[...]
    so = so_ref[...]

    def row64(bref):
        return jnp.concatenate([bref[g:g + 1, :] for g in range(_G)], axis=1)

    bexp_row = row64(bexp_ref)
    brows = (row64(bq_ref), row64(bk_ref), row64(bv_ref))
    for j, (wref, brow) in enumerate(((wq_ref, brows[0]), (wk_ref, brows[1]),
                                      (wv_ref, brows[2]))):
        wf = wref[...]
        bd = jnp.concatenate([wf, wf, wf, wf], axis=1) * gm
        prod = _dot(bd, wexp)
        crow = _dot_t(bexp_row, bd) + brow
        base = j * 144
        biga_ref[base:base + 64, :] = _dot_tl(prod, se)
        biga_ref[base + 64:base + 65, :] = _dot(crow, se)
        biga_ref[base + 72:base + 136, :] = _dot_tl(prod, so)
        biga_ref[base + 136:base + 137, :] = _dot(crow, so)
    wo = wo_ref[...]
    bdo = jnp.concatenate([wo, wo, wo, wo], axis=1) * gm
    def selt_bdot(sel_mat):
        return jax.lax.dot_general(sel_mat, bdo, (((0,), (1,)), ((), ())),
                                   preferred_element_type=_f32)

    bigwo_ref[0:32, :] = selt_bdot(se)
    bigwo_ref[32:33, :] = row64(bo_ref)
    bigwo_ref[40:72, :] = selt_bdot(so)


def _body(price_ref, news_ref, maskt_ref, wr_ref, br_ref, wgt_ref, bg_ref,
          biga_ref, bigwo_ref, segst_ref, segc_ref,
          pred_ref, rwt_ref, ht_ref, idxt_ref):
    b = price_ref.shape[0]
    mask = maskt_ref[...].T
    x = price_ref[...] + news_ref[...] * mask[:, :, None]
    pooled = jnp.sum(x, axis=1) * (1.0 / _T)
    hr = jnp.tanh(_dot(pooled, wr_ref[...]) + br_ref[...])
    h_row = _dot_t(hr, wgt_ref[...]) + bg_ref[...]
    ht = h_row.T
    ht_ref[...] = ht

    ones_row = jnp.ones((1, b), _f32)
    h_aug = jnp.concatenate([ht, ones_row], axis=0)

    def proj(base):
        return _dot_tl(biga_ref[base:base + 65, :], h_aug)

    qe = proj(0)
    qo = proj(72)
    ke = proj(144)
    ko = proj(216)
    ve = proj(288)
    vo = proj(360)
    segst = segst_ref[...]
    segc = segc_ref[...]
    p00 = _dot(segst, qe * ke)
    p01 = _dot(segst, qe * ko)
    p10 = _dot(segst, qo * ke)
    p11 = _dot(segst, qo * ko)
    m0 = jnp.maximum(p00, p01)
    e00 = jnp.exp(p00 - m0)
    e01 = jnp.exp(p01 - m0)
    m1 = jnp.maximum(p10, p11)
    e10 = jnp.exp(p10 - m1)
    e11 = jnp.exp(p11 - m1)
    z0 = e00 + e01
    z1 = e10 + e11
    av_e = _dot(segc, e00 / z0) * ve + _dot(segc, e01 / z0) * vo
    av_o = _dot(segc, e10 / z1) * ve + _dot(segc, e11 / z1) * vo
    av_e_aug = jnp.concatenate([av_e, ones_row], axis=0)
    agg = (_dot_tl(bigwo_ref[0:33, :], av_e_aug) +
           _dot_tl(bigwo_ref[40:72, :], av_o))

    iota = jax.lax.broadcasted_iota(jnp.int32, ht.shape, 0)
    cur = ht
    sel = jnp.zeros(ht.shape, jnp.bool_)
    mtop = None
    idx_rows = []
    for k in range(_K):
        m = jnp.max(cur, axis=0, keepdims=True)
        if k == 0:
            mtop = m
        idx = jnp.min(jnp.where(cur == m, iota, _HID), axis=0, keepdims=True)
        idx_rows.append(idx)
        chosen = iota == idx
        sel = jnp.logical_or(sel, chosen)
        cur = jnp.where(chosen, -jnp.inf, cur)
    idxt_ref[...] = jnp.concatenate(idx_rows, axis=0)
    ex = jnp.where(sel, jnp.exp(ht - mtop), 0.0)
    rwt = ex / jnp.sum(ex, axis=0, keepdims=True)
    rwt_ref[...] = rwt
    pred_ref[...] = jnp.sum(agg * rwt, axis=0)


def kernel(price_feature, news_feature, news_mask, W_r, b_r, W_g, b_g,
           W_exp, b_exp, Wq, bq, Wk, bk, Wv, bv, Wo, bo):
    n = price_feature.shape[0]

    se = np.zeros((_HID, _G * _NH), np.float32)
    so = np.zeros((_HID, _G * _NH), np.float32)
    seg = np.zeros((_G * _NH, _G), np.float32)
    for g in range(_G):
        for hh in range(_NH):
            se[g * _EPG + 2 * hh, g * _NH + hh] = 1.0
            so[g * _EPG + 2 * hh + 1, g * _NH + hh] = 1.0
            seg[g * _NH + hh, g] = 1.0
    gm = np.zeros((_HID, _HID), np.float32)
    for g in range(_G):
        gm[g * _EPG:(g + 1) * _EPG, g * _EPG:(g + 1) * _EPG] = 1.0
    segst = jnp.asarray(seg.T / np.sqrt(np.float32(_HD)))
    segc = jnp.asarray(seg)

    def full(shape):
        return pl.BlockSpec(shape, lambda i: tuple(0 for _ in shape))

    biga, bigwo = pl.pallas_call(
        _prep_body,
        grid=(1,),
        in_specs=[full((_HID, _EPG))] * 4 + [
            full((_HID, _HID)),
            full((_G, _EPG)), full((_G, _EPG)), full((_G, _EPG)),
            full((_G, _EPG)), full((_G, _EPG)),
            full((_HID, _HID)),
            full((_HID, 32)), full((_HID, 32)),
        ],
        out_specs=(full((432, 32)), full((72, _HID))),
        out_shape=(
            jax.ShapeDtypeStruct((432, 32), _f32),
            jax.ShapeDtypeStruct((72, _HID), _f32),
        ),
    )(Wq.reshape(_HID, _EPG), Wk.reshape(_HID, _EPG),
      Wv.reshape(_HID, _EPG), Wo.reshape(_HID, _EPG),
      W_exp.reshape(_HID, _HID), b_exp, bq, bk, bv, bo,
      jnp.asarray(gm), jnp.asarray(se), jnp.asarray(so))

    b = _B if n % _B == 0 else n
    grid = (n // b,)

    outs = pl.pallas_call(
        _body,
        grid=grid,
        in_specs=[
            pl.BlockSpec((b, _T, _D), lambda i: (i, 0, 0)),
            pl.BlockSpec((b, _T, _D), lambda i: (i, 0, 0)),
            pl.BlockSpec((_T, b), lambda i: (0, i)),
            full((_D, _DG)), full((1, _DG)),
            full((_HID, _DG)), full((1, _HID)),
            full((432, 32)), full((72, _HID)),
            full((_G, _G * _NH)), full((_G * _NH, _G)),
        ],
        out_specs=(
            pl.BlockSpec((b,), lambda i: (i,)),
            pl.BlockSpec((_HID, b), lambda i: (0, i)),
            pl.BlockSpec((_HID, b), lambda i: (0, i)),
            pl.BlockSpec((_K, b), lambda i: (0, i)),
        ),
        out_shape=(
            jax.ShapeDtypeStruct((n,), _f32),
            jax.ShapeDtypeStruct((_HID, n), _f32),
            jax.ShapeDtypeStruct((_HID, n), _f32),
            jax.ShapeDtypeStruct((_K, n), jnp.int32),
        ),
        compiler_params=pltpu.CompilerParams(
            dimension_semantics=("arbitrary",),
            vmem_limit_bytes=100 * 1024 * 1024),
    )(price_feature, news_feature, news_mask.T,
      W_r, b_r.reshape(1, -1), W_g.T, b_g.reshape(1, -1),
      biga, bigwo, segst, segc)

    preds, rwt, ht, idxt = outs
    rw = rwt.T
    return preds, rw, ht.T, idxt.T, rw


# speedup vs baseline: 1.8169x; 1.0413x over previous
import jax
import jax.numpy as jnp
import numpy as np
from jax.experimental import pallas as pl
from jax.experimental.pallas import tpu as pltpu

_T, _D = 16, 128
_DG = 128
_G, _EPG = 4, 16
_HID = _G * _EPG
_NH = 8
_HD = _EPG // _NH
_K = 8
_B = 1024

_f32 = jnp.float32


def _dot(a, b):
    return jnp.dot(a, b, preferred_element_type=_f32)


def _dot_t(a, b):
    return jax.lax.dot_general(a, b, (((1,), (1,)), ((), ())),
                               preferred_element_type=_f32)


def _dot_tl(a, b):
    return jax.lax.dot_general(a, b, (((0,), (0,)), ((), ())),
                               preferred_element_type=_f32)


def _prep_body(wq_ref, wk_ref, wv_ref, wo_ref, wexp_ref, bexp_ref,
               bq_ref, bk_ref, bv_ref, bo_ref, gm_ref, se_ref, so_ref,
               biga_ref, bigwo_ref):
    wexp = wexp_ref[...]
    gm = gm_ref[...]
    se = se_ref[...]
    so = so_ref[...]

    def row64(bref):
        return jnp.concatenate([bref[g:g + 1, :] for g in range(_G)], axis=1)

    bexp_row = row64(bexp_ref)
    brows = (row64(bq_ref), row64(bk_ref), row64(bv_ref))
    for j, (wref, brow) in enumerate(((wq_ref, brows[0]), (wk_ref, brows[1]),
                                      (wv_ref, brows[2]))):
        wf = wref[...]
        bd = jnp.concatenate([wf, wf, wf, wf], axis=1) * gm
        prod = _dot(bd, wexp)
        crow = _dot_t(bexp_row, bd) + brow
        base = j * 144
        biga_ref[base:base + 64, :] = _dot_tl(prod, se)
        biga_ref[base + 64:base + 65, :] = _dot(crow, se)
        biga_ref[base + 72:base + 136, :] = _dot_tl(prod, so)
        biga_ref[base + 136:base + 137, :] = _dot(crow, so)
    wo = wo_ref[...]
    bdo = jnp.concatenate([wo, wo, wo, wo], axis=1) * gm
    def selt_bdot(sel_mat):
        return jax.lax.dot_general(sel_mat, bdo, (((0,), (1,)), ((), ())),
                                   preferred_element_type=_f32)

    bigwo_ref[0:32, :] = selt_bdot(se)
    bigwo_ref[32:33, :] = row64(bo_ref)
    bigwo_ref[40:72, :] = selt_bdot(so)


def _body(price_ref, news_ref, maskt_ref, wr_ref, br_ref, wgt_ref, bg_ref,
          biga_ref, bigwo_ref, segst_ref, segc_ref,
          pred_ref, rwt_ref, ht_ref, idxt_ref, rwt2_ref):
    b = price_ref.shape[0]
    mask = maskt_ref[...].T
    x = price_ref[...] + news_ref[...] * mask[:, :, None]
    pooled = jnp.sum(x, axis=1) * (1.0 / _T)
    hr = jnp.tanh(_dot(pooled, wr_ref[...]) + br_ref[...])
    h_row = _dot_t(hr, wgt_ref[...]) + bg_ref[...]
    ht = h_row.T
    ht_ref[...] = ht

    ones_row = jnp.ones((1, b), _f32)
    h_aug = jnp.concatenate([ht, ones_row], axis=0)

    def proj(base):
        return _dot_tl(biga_ref[base:base + 65, :], h_aug)

    qe = proj(0)
    qo = proj(72)
    ke = proj(144)
    ko = proj(216)
    ve = proj(288)
    vo = proj(360)
    segst = segst_ref[...]
    segc = segc_ref[...]
    p00 = _dot(segst, qe * ke)
    p01 = _dot(segst, qe * ko)
    p10 = _dot(segst, qo * ke)
    p11 = _dot(segst, qo * ko)
    m0 = jnp.maximum(p00, p01)
    e00 = jnp.exp(p00 - m0)
    e01 = jnp.exp(p01 - m0)
    m1 = jnp.maximum(p10, p11)
    e10 = jnp.exp(p10 - m1)
    e11 = jnp.exp(p11 - m1)
    z0 = e00 + e01
    z1 = e10 + e11
    av_e = _dot(segc, e00 / z0) * ve + _dot(segc, e01 / z0) * vo
    av_o = _dot(segc, e10 / z1) * ve + _dot(segc, e11 / z1) * vo
    av_e_aug = jnp.concatenate([av_e, ones_row], axis=0)
    agg = (_dot_tl(bigwo_ref[0:33, :], av_e_aug) +
           _dot_tl(bigwo_ref[40:72, :], av_o))

    iota = jax.lax.broadcasted_iota(jnp.int32, ht.shape, 0)
    cur = ht
    sel = jnp.zeros(ht.shape, jnp.bool_)
    mtop = None
    idx_rows = []
    for k in range(_K):
        m = jnp.max(cur, axis=0, keepdims=True)
        if k == 0:
            mtop = m
        idx = jnp.min(jnp.where(cur == m, iota, _HID), axis=0, keepdims=True)
        idx_rows.append(idx)
        chosen = iota == idx
        sel = jnp.logical_or(sel, chosen)
        cur = jnp.where(chosen, -jnp.inf, cur)
    idxt_ref[...] = jnp.concatenate(idx_rows, axis=0)
    ex = jnp.where(sel, jnp.exp(ht - mtop), 0.0)
    rwt = ex / jnp.sum(ex, axis=0, keepdims=True)
    rwt_ref[...] = rwt
    rwt2_ref[...] = rwt
    pred_ref[...] = jnp.sum(agg * rwt, axis=0)


def kernel(price_feature, news_feature, news_mask, W_r, b_r, W_g, b_g,
           W_exp, b_exp, Wq, bq, Wk, bk, Wv, bv, Wo, bo):
    n = price_feature.shape[0]

    se = np.zeros((_HID, _G * _NH), np.float32)
    so = np.zeros((_HID, _G * _NH), np.float32)
    seg = np.zeros((_G * _NH, _G), np.float32)
    for g in range(_G):
        for hh in range(_NH):
            se[g * _EPG + 2 * hh, g * _NH + hh] = 1.0
            so[g * _EPG + 2 * hh + 1, g * _NH + hh] = 1.0
            seg[g * _NH + hh, g] = 1.0
    gm = np.zeros((_HID, _HID), np.float32)
    for g in range(_G):
        gm[g * _EPG:(g + 1) * _EPG, g * _EPG:(g + 1) * _EPG] = 1.0
    segst = jnp.asarray(seg.T / np.sqrt(np.float32(_HD)))
    segc = jnp.asarray(seg)

    def full(shape):
        return pl.BlockSpec(shape, lambda i: tuple(0 for _ in shape))

    biga, bigwo = pl.pallas_call(
        _prep_body,
        grid=(1,),
        in_specs=[full((_HID, _EPG))] * 4 + [
            full((_HID, _HID)),
            full((_G, _EPG)), full((_G, _EPG)), full((_G, _EPG)),
            full((_G, _EPG)), full((_G, _EPG)),
            full((_HID, _HID)),
            full((_HID, 32)), full((_HID, 32)),
        ],
        out_specs=(full((432, 32)), full((72, _HID))),
        out_shape=(
            jax.ShapeDtypeStruct((432, 32), _f32),
            jax.ShapeDtypeStruct((72, _HID), _f32),
        ),
    )(Wq.reshape(_HID, _EPG), Wk.reshape(_HID, _EPG),
      Wv.reshape(_HID, _EPG), Wo.reshape(_HID, _EPG),
      W_exp.reshape(_HID, _HID), b_exp, bq, bk, bv, bo,
      jnp.asarray(gm), jnp.asarray(se), jnp.asarray(so))

    b = _B if n % _B == 0 else n
    grid = (n // b,)

    outs = pl.pallas_call(
        _body,
        grid=grid,
        in_specs=[
            pl.BlockSpec((b, _T, _D), lambda i: (i, 0, 0)),
            pl.BlockSpec((b, _T, _D), lambda i: (i, 0, 0)),
            pl.BlockSpec((_T, b), lambda i: (0, i)),
            full((_D, _DG)), full((1, _DG)),
            full((_HID, _DG)), full((1, _HID)),
            full((432, 32)), full((72, _HID)),
            full((_G, _G * _NH)), full((_G * _NH, _G)),
        ],
        out_specs=(
            pl.BlockSpec((b,), lambda i: (i,)),
            pl.BlockSpec((_HID, b), lambda i: (0, i)),
            pl.BlockSpec((_HID, b), lambda i: (0, i)),
            pl.BlockSpec((_K, b), lambda i: (0, i)),
            pl.BlockSpec((_HID, b), lambda i: (0, i)),
        ),
        out_shape=(
            jax.ShapeDtypeStruct((n,), _f32),
            jax.ShapeDtypeStruct((_HID, n), _f32),
            jax.ShapeDtypeStruct((_HID, n), _f32),
            jax.ShapeDtypeStruct((_K, n), jnp.int32),
            jax.ShapeDtypeStruct((_HID, n), _f32),
        ),
        compiler_params=pltpu.CompilerParams(
            dimension_semantics=("arbitrary",),
            vmem_limit_bytes=100 * 1024 * 1024),
    )(price_feature, news_feature, news_mask.T,
      W_r, b_r.reshape(1, -1), W_g.T, b_g.reshape(1, -1),
      biga, bigwo, segst, segc)

    preds, rwt, ht, idxt, rwt2 = outs
    return preds, rwt.T, ht.T, idxt.T, rwt2.T
